# baseline (speedup 1.0000x reference)
import functools

import jax
import jax.numpy as jnp
from jax import lax
from jax.experimental import pallas as pl
from jax.experimental.pallas import tpu as pltpu
from jax.experimental.pallas import tpu_sc as plsc


def _layer_norm(h, g, b, eps=1e-5):
    mu = jnp.mean(h, axis=-1, keepdims=True)
    var = jnp.mean((h - mu) ** 2, axis=-1, keepdims=True)
    return (h - mu) / jnp.sqrt(var + eps) * g + b


def _gelu_exact(h):
    return 0.5 * h * (1.0 + jax.lax.erf(h * (2.0 ** -0.5)))


def _mlp_kernel(x_ref, w1_ref, b1_ref, g1_ref, be1_ref, w2_ref, b2_ref,
                g2_ref, be2_ref, w3_ref, b3_ref, t_ref, logits_ref):
    dn = (((1,), (1,)), ((), ()))
    x = x_ref[...]
    h = jax.lax.dot_general(x, w1_ref[...], dn, preferred_element_type=jnp.float32) + b1_ref[...]
    h = _gelu_exact(_layer_norm(h, g1_ref[...], be1_ref[...]))
    h = jax.lax.dot_general(h, w2_ref[...], dn, preferred_element_type=jnp.float32) + b2_ref[...]
    h = _gelu_exact(_layer_norm(h, g2_ref[...], be2_ref[...]))
    logits = jax.lax.dot_general(h, w3_ref[...], dn, preferred_element_type=jnp.float32) + b3_ref[...]
    t = jnp.maximum(t_ref[0, 0], 0.1)
    logits_ref[...] = logits / t


def _mlp_logits(x, W1, b1, ln1_g, ln1_b, W2, b2, ln2_g, ln2_b, W3, b3,
                temperature):
    N, D = x.shape
    H = W1.shape[0]
    H2 = W2.shape[0]
    E = W3.shape[0]
    B = min(4096, N)
    grid = (N // B,)

    row_spec = lambda shape: pl.BlockSpec(shape, lambda i: (i, 0))
    full_spec = lambda shape: pl.BlockSpec(shape, lambda i: (0, 0))

    return pl.pallas_call(
        _mlp_kernel,
        grid=grid,
        in_specs=[
            row_spec((B, D)),
            full_spec((H, D)), full_spec((1, H)), full_spec((1, H)), full_spec((1, H)),
            full_spec((H2, H)), full_spec((1, H2)), full_spec((1, H2)), full_spec((1, H2)),
            full_spec((E, H2)), full_spec((1, E)),
            pl.BlockSpec(memory_space=pltpu.SMEM),
        ],
        out_specs=row_spec((B, E)),
        out_shape=jax.ShapeDtypeStruct((N, E), jnp.float32),
    )(
        x,
        W1, b1.reshape(1, H), ln1_g.reshape(1, H), ln1_b.reshape(1, H),
        W2, b2.reshape(1, H2), ln2_g.reshape(1, H2), ln2_b.reshape(1, H2),
        W3, b3.reshape(1, E),
        temperature.reshape(1, 1),
    )


_NC = 2
_NS = 16
_L = 16


def _route_body(N, E, C, logits_hbm, gates_hbm, idx_hbm, lv, gv, iv):
    tpw = N // (_NC * _NS)
    wid = lax.axis_index("s") * _NC + lax.axis_index("c")
    base = wid * tpw
    lane = lax.iota(jnp.int32, _L)
    zf = jnp.zeros((_L,), jnp.float32)
    zi = jnp.zeros((_L,), jnp.int32)
    ninf = jnp.full((_L,), -jnp.inf, jnp.float32)

    def _zrow(r, carry):
        gv[pl.ds(r * _L, _L)] = zf
        return carry
    lax.fori_loop(0, C * E // _L, _zrow, 0)

    for c in range(tpw // C):
        tbase = base + c * C
        pltpu.sync_copy(logits_hbm.at[pl.ds(tbase * E, C * E)], lv)
        written = []
        for g in range(C // _L):
            toks = lane + (g * _L)
            tE = toks * E

            def _step(e, carry):
                m1, i1, m2, i2 = carry
                ev = zi + e
                v = plsc.load_gather(lv, [tE + ev])
                gt1 = v > m1
                gt2 = jnp.logical_and(v > m2, jnp.logical_not(gt1))
                i2 = jnp.where(gt1, i1, jnp.where(gt2, ev, i2))
                m2 = jnp.where(gt1, m1, jnp.where(gt2, v, m2))
                i1 = jnp.where(gt1, ev, i1)
                m1 = jnp.where(gt1, v, m1)
                return m1, i1, m2, i2

            m1, i1, m2, i2 = lax.fori_loop(0, E, _step, (ninf, zi, ninf, zi))

            e2 = jnp.exp(m2 - m1)
            denom = 1.0 + e2
            g1 = 1.0 / denom
            g2 = e2 / denom
            s = g1 + g2 + 1e-8
            g1 = g1 / s
            g2 = g2 / s

            plsc.store_scatter(gv, [tE + i1], g1)
            plsc.store_scatter(gv, [tE + i2], g2)
            plsc.store_scatter(iv, [toks * 2], i1)
            plsc.store_scatter(iv, [toks * 2 + 1], i2)
            written.append((tE, i1, i2))

        pltpu.sync_copy(gv, gates_hbm.at[pl.ds(tbase * E, C * E)])
        pltpu.sync_copy(iv, idx_hbm.at[pl.ds(tbase * 2, C * 2)])
        for tE, i1, i2 in written:
            plsc.store_scatter(gv, [tE + i1], zf)
            plsc.store_scatter(gv, [tE + i2], zf)


def _route_sc(logits):
    N, E = logits.shape
    C = min(256, N // (_NC * _NS))
    mesh = plsc.VectorSubcoreMesh(core_axis_name="c", subcore_axis_name="s")
    gates_flat, idx_flat = pl.kernel(
        functools.partial(_route_body, N, E, C),
        out_type=[
            jax.ShapeDtypeStruct((N * E,), jnp.float32),
            jax.ShapeDtypeStruct((N * 2,), jnp.int32),
        ],
        mesh=mesh,
        compiler_params=pltpu.CompilerParams(needs_layout_passes=False),
        scratch_types=[
            pltpu.VMEM((C * E,), jnp.float32),
            pltpu.VMEM((C * E,), jnp.float32),
            pltpu.VMEM((C * 2,), jnp.int32),
        ],
    )(logits.reshape(N * E))
    return gates_flat.reshape(N, E), idx_flat.reshape(N, 2)


def kernel(x, W1, b1, ln1_g, ln1_b, W2, b2, ln2_g, ln2_b, W3, b3, temperature):
    logits = _mlp_logits(x, W1, b1, ln1_g, ln1_b, W2, b2, ln2_g, ln2_b,
                         W3, b3, temperature)
    gates, idx = _route_sc(logits)
    return gates, idx, logits


# speedup vs baseline: 1.8647x; 1.8647x over previous
import functools

import jax
import jax.numpy as jnp
from jax.experimental import pallas as pl
from jax.experimental.pallas import tpu as pltpu


def _layer_norm(h, g, b, eps=1e-5):
    mu = jnp.mean(h, axis=-1, keepdims=True)
    var = jnp.mean((h - mu) ** 2, axis=-1, keepdims=True)
    return (h - mu) / jnp.sqrt(var + eps) * g + b


def _gelu_exact(h):
    return 0.5 * h * (1.0 + jax.lax.erf(h * (2.0 ** -0.5)))


def _gate_kernel(x_ref, w1_ref, b1_ref, g1_ref, be1_ref, w2_ref, b2_ref,
                 g2_ref, be2_ref, w3_ref, b3_ref, t_ref,
                 gates_ref, idx_ref, logits_ref):
    dn = (((1,), (1,)), ((), ()))
    x = x_ref[...]
    h = jax.lax.dot_general(x, w1_ref[...], dn, preferred_element_type=jnp.float32) + b1_ref[...]
    h = _gelu_exact(_layer_norm(h, g1_ref[...], be1_ref[...]))
    h = jax.lax.dot_general(h, w2_ref[...], dn, preferred_element_type=jnp.float32) + b2_ref[...]
    h = _gelu_exact(_layer_norm(h, g2_ref[...], be2_ref[...]))
    logits = jax.lax.dot_general(h, w3_ref[...], dn, preferred_element_type=jnp.float32) + b3_ref[...]
    t = jnp.maximum(t_ref[0, 0], 0.1)
    logits = logits / t

    B, E = logits.shape
    col = jax.lax.broadcasted_iota(jnp.int32, (B, E), 1).astype(jnp.float32)
    rev = (E - 1.0) - col
    m1 = jnp.max(logits, axis=-1, keepdims=True)
    a1 = jnp.max(jnp.where(logits == m1, rev, -1.0), axis=-1, keepdims=True)
    i1 = (E - 1.0) - a1
    masked = jnp.where(col == i1, -jnp.inf, logits)
    m2 = jnp.max(masked, axis=-1, keepdims=True)
    a2 = jnp.max(jnp.where(masked == m2, rev, -1.0), axis=-1, keepdims=True)
    i2 = (E - 1.0) - a2

    e2 = jnp.exp(m2 - m1)
    denom = 1.0 + e2
    g1 = 1.0 / denom
    g2 = e2 / denom
    s = g1 + g2 + 1e-8
    g1 = g1 / s
    g2 = g2 / s

    gates_ref[...] = jnp.where(col == i1, g1, 0.0) + jnp.where(col == i2, g2, 0.0)
    logits_ref[...] = logits
    idxcol = jax.lax.broadcasted_iota(jnp.int32, idx_ref.shape, 1)
    idx_ref[...] = jnp.where(idxcol == 0, i1, i2).astype(jnp.int32)


def kernel(x, W1, b1, ln1_g, ln1_b, W2, b2, ln2_g, ln2_b, W3, b3, temperature):
    N, D = x.shape
    H = W1.shape[0]
    H2 = W2.shape[0]
    E = W3.shape[0]
    B = min(4096, N)
    grid = (N // B,)
    IPAD = 2

    row_spec = lambda shape: pl.BlockSpec(shape, lambda i: (i, 0))
    full_spec = lambda shape: pl.BlockSpec(shape, lambda i: (0, 0))

    gates, idx_pad, logits = pl.pallas_call(
        _gate_kernel,
        grid=grid,
        compiler_params=pltpu.CompilerParams(
            dimension_semantics=("parallel",)),
        in_specs=[
            row_spec((B, D)),
            full_spec((H, D)), full_spec((1, H)), full_spec((1, H)), full_spec((1, H)),
            full_spec((H2, H)), full_spec((1, H2)), full_spec((1, H2)), full_spec((1, H2)),
            full_spec((E, H2)), full_spec((1, E)),
            pl.BlockSpec(memory_space=pltpu.SMEM),
        ],
        out_specs=[
            row_spec((B, E)),
            row_spec((B, IPAD)),
            row_spec((B, E)),
        ],
        out_shape=[
            jax.ShapeDtypeStruct((N, E), jnp.float32),
            jax.ShapeDtypeStruct((N, IPAD), jnp.int32),
            jax.ShapeDtypeStruct((N, E), jnp.float32),
        ],
    )(
        x,
        W1, b1.reshape(1, H), ln1_g.reshape(1, H), ln1_b.reshape(1, H),
        W2, b2.reshape(1, H2), ln2_g.reshape(1, H2), ln2_b.reshape(1, H2),
        W3, b3.reshape(1, E),
        temperature.reshape(1, 1),
    )
    return gates, idx_pad, logits
